# Initial kernel scaffold; baseline (speedup 1.0000x reference)
import jax
import jax.numpy as jnp
from jax.experimental import pallas as pl


def kernel(x, U0, U1, U2, U3):
    raise NotImplementedError("write your pallas kernel here")


# speedup vs baseline: 5.0730x; 5.0730x over previous
import functools

import jax
import jax.numpy as jnp
from jax import lax
from jax.experimental import pallas as pl
from jax.experimental.pallas import tpu as pltpu
from jax.experimental.pallas import tpu_sc as plsc

_B, _F = 16384, 26
_OUT = 64
_N = _B * _F
_NW = 32
_PER_W = _N // _NW
_CHUNK = 128
_NCHUNK = _PER_W // _CHUNK
_G = 16
_NG = _CHUNK // _G


def _prep_body(u0, u1, u2, u3, t1, t2):
    t1[...] = jnp.dot(u0[...], u1[...], preferred_element_type=jnp.float32)
    t2[...] = jnp.dot(u2[...], u3[...], preferred_element_type=jnp.float32)


def _prep(U0, U1, U2, U3):
    return pl.pallas_call(
        _prep_body,
        out_shape=(
            jax.ShapeDtypeStruct((160, 800), jnp.float32),
            jax.ShapeDtypeStruct((640, 50), jnp.float32),
        ),
    )(U0.reshape(160, 8), U1.reshape(8, 800),
      U2.reshape(640, 8), U3.reshape(8, 50))


_sc_mesh = plsc.VectorSubcoreMesh(core_axis_name="c", subcore_axis_name="s")


@functools.partial(
    pl.kernel,
    out_type=jax.ShapeDtypeStruct((_N * _OUT,), jnp.float32),
    mesh=_sc_mesh,
    scratch_types=[
        pltpu.VMEM((_CHUNK,), jnp.int32),
        pltpu.VMEM((_CHUNK,), jnp.int32),
        pltpu.VMEM((_CHUNK,), jnp.int32),
        pltpu.VMEM((_CHUNK,), jnp.int32),
        pltpu.VMEM((_CHUNK,), jnp.int32),
        pltpu.VMEM((_CHUNK,), jnp.int32),
        pltpu.VMEM((_CHUNK, 128), jnp.float32),
        pltpu.VMEM((_CHUNK, 128), jnp.float32),
        pltpu.VMEM((32000,), jnp.float32),
        pltpu.VMEM((_CHUNK * _OUT,), jnp.float32),
        pltpu.VMEM((_CHUNK * _OUT,), jnp.float32),
        pltpu.SemaphoreType.DMA,
        pltpu.SemaphoreType.DMA,
        pltpu.SemaphoreType.DMA,
        pltpu.SemaphoreType.DMA,
    ],
    compiler_params=pltpu.CompilerParams(needs_layout_passes=False),
)
def _sc_lookup(x_hbm, a2_hbm, bt_hbm, out_hbm,
               x_b0, x_b1, r_b0, r_b1, c_b0, c_b1,
               a_b0, a_b1, bt_vmem, o_b0, o_b1,
               sem_a0, sem_a1, sem_o0, sem_o1):
    x_buf = (x_b0, x_b1)
    ridx = (r_b0, r_b1)
    cidx = (c_b0, c_b1)
    a_buf = (a_b0, a_b1)
    o_buf = (o_b0, o_b1)
    sem_a = (sem_a0, sem_a1)
    sem_o = (sem_o0, sem_o1)
    wid = lax.axis_index("s") * 2 + lax.axis_index("c")
    base = wid * _PER_W
    iota = lax.iota(jnp.int32, _G)
    div1000 = jnp.full((_G,), 1000, jnp.int32)

    def _prefetch(k, nb):
        off = base + k * _CHUNK
        pltpu.sync_copy(x_hbm.at[pl.ds(off, _CHUNK)], x_buf[nb])

        def idx_body(g, carry):
            xv = x_buf[nb][pl.ds(g * _G, _G)]
            r = lax.div(xv, div1000)
            ridx[nb][pl.ds(g * _G, _G)] = r
            cidx[nb][pl.ds(g * _G, _G)] = xv - r * 1000
            return carry

        lax.fori_loop(0, _NG, idx_body, 0)
        pltpu.make_async_copy(a2_hbm.at[ridx[nb]], a_buf[nb], sem_a[nb]).start()

    def _compute(b):
        pltpu.make_async_copy(a2_hbm.at[ridx[b]], a_buf[b], sem_a[b]).wait()

        def group_body(g, carry):
            tvec = iota + g * _G
            ovec = tvec * _OUT
            cb = cidx[b][pl.ds(g * _G, _G)] * 32
            bg = [[plsc.load_gather(bt_vmem, [cb + (c8 * 4 + d)])
                   for d in range(4)] for c8 in range(8)]
            for a in range(16):
                av = [plsc.load_gather(
                          a_buf[b],
                          [tvec, jnp.full((_G,), c8 * 16 + a, jnp.int32)])
                      for c8 in range(8)]
                for d in range(4):
                    acc = av[0] * bg[0][d]
                    for c8 in range(1, 8):
                        acc = acc + av[c8] * bg[c8][d]
                    plsc.store_scatter(o_buf[b], [ovec + (a * 4 + d)], acc)
            return carry

        lax.fori_loop(0, _NG, group_body, 0)

    def _flush(k, b):
        off = (base + k * _CHUNK) * _OUT
        pltpu.make_async_copy(o_buf[b], out_hbm.at[pl.ds(off, _CHUNK * _OUT)],
                              sem_o[b]).start()

    def _drain(b):
        pltpu.make_async_copy(o_buf[b], out_hbm.at[pl.ds(0, _CHUNK * _OUT)],
                              sem_o[b]).wait()

    pltpu.sync_copy(bt_hbm, bt_vmem)
    _prefetch(0, 0)

    def body(i, carry):
        k0 = i * 2
        for bb in range(2):
            k = k0 + bb
            nb = (bb + 1) % 2
            if bb == 0:
                _prefetch(k + 1, nb)
            else:
                @pl.when(k + 1 < _NCHUNK)
                def _():
                    _prefetch(k + 1, nb)

            @pl.when(k0 >= 2)
            def _():
                _drain(bb)

            _compute(bb)
            _flush(k, bb)
        return carry

    lax.fori_loop(0, _NCHUNK // 2, body, 0)
    _drain(0)
    _drain(1)


def kernel(x, U0, U1, U2, U3):
    t1, t2 = _prep(U0, U1, U2, U3)
    a2 = t1.reshape(40, 4, 25, 4, 8).transpose(0, 2, 4, 1, 3).reshape(1000, 128)
    bt = t2.reshape(8, 40, 2, 25, 2).transpose(1, 3, 0, 2, 4).reshape(32000)
    out = _sc_lookup(x.reshape(_N), a2, bt)
    return out.reshape(_B, _F, _OUT)


# speedup vs baseline: 8.3644x; 1.6488x over previous
import functools

import jax
import jax.numpy as jnp
from jax import lax
from jax.experimental import pallas as pl
from jax.experimental.pallas import tpu as pltpu
from jax.experimental.pallas import tpu_sc as plsc

_B, _F = 16384, 26
_OUT = 64
_N = _B * _F
_NW = 32
_PER_W = _N // _NW
_CHUNK = 128
_NCHUNK = _PER_W // _CHUNK
_G = 16
_NG = _CHUNK // _G


def _prep_body(u0, u1, u2, u3, t1, t2):
    t1[...] = jnp.dot(u0[...], u1[...], preferred_element_type=jnp.float32)
    t2[...] = jnp.dot(u2[...], u3[...], preferred_element_type=jnp.float32)


def _prep(U0, U1, U2, U3):
    return pl.pallas_call(
        _prep_body,
        out_shape=(
            jax.ShapeDtypeStruct((160, 800), jnp.float32),
            jax.ShapeDtypeStruct((640, 50), jnp.float32),
        ),
    )(U0.reshape(160, 8), U1.reshape(8, 800),
      U2.reshape(640, 8), U3.reshape(8, 50))


_sc_mesh = plsc.VectorSubcoreMesh(core_axis_name="c", subcore_axis_name="s")


@functools.partial(
    pl.kernel,
    out_type=jax.ShapeDtypeStruct((_N * _OUT,), jnp.float32),
    mesh=_sc_mesh,
    scratch_types=[
        pltpu.VMEM((64000,), jnp.int32),
        pltpu.VMEM((32000,), jnp.float32),
        pltpu.VMEM((_PER_W,), jnp.int32),
        pltpu.VMEM((_CHUNK * _OUT,), jnp.float32),
        pltpu.VMEM((_CHUNK * _OUT,), jnp.float32),
        pltpu.SemaphoreType.DMA,
        pltpu.SemaphoreType.DMA,
    ],
    compiler_params=pltpu.CompilerParams(
        needs_layout_passes=False, disable_bounds_checks=True),
)
def _sc_lookup(x_hbm, t1p_hbm, t2_hbm, out_hbm,
               t1p, t2, x_all, o_b0, o_b1, sem_o0, sem_o1):
    o_buf = (o_b0, o_b1)
    sem_o = (sem_o0, sem_o1)
    wid = lax.axis_index("s") * 2 + lax.axis_index("c")
    base = wid * _PER_W
    iota = lax.iota(jnp.int32, _G)

    pltpu.sync_copy(t1p_hbm, t1p)
    pltpu.sync_copy(t2_hbm, t2)
    pltpu.sync_copy(x_hbm.at[pl.ds(base, _PER_W)], x_all)

    def _fdiv(v, recip, mul):
        q = ((v.astype(jnp.float32) + 0.5) * recip).astype(jnp.int32)
        return q, v - q * mul

    def _compute(k, b):
        def group_body(g, carry):
            xv = x_all[pl.ds(k * _CHUNK + g * _G, _G)]
            ovec = (iota + g * _G) * _OUT
            rv, cv = _fdiv(xv, 0.001, 1000)
            n1, n2 = _fdiv(rv, 0.04, 25)
            n3, n4 = _fdiv(cv, 0.04, 25)
            base_a = n1 * 1600 + n2 * 16
            base_b = n3 * 100 + n4 * 2
            bg = [[plsc.load_gather(
                       t2, [base_b + (c8 * 4000 + m3 * 50 + m4)])
                   for m3 in range(2) for m4 in range(2)]
                  for c8 in range(8)]
            for m1 in range(4):
                for m2 in range(4):
                    a = m1 * 4 + m2
                    av = []
                    for cp in range(4):
                        packed = plsc.load_gather(
                            t1p, [base_a + (m1 * 400 + m2 * 4 + cp)])
                        lo, hi = plsc.unpack(
                            plsc.bitcast(packed, jnp.bfloat16),
                            format=plsc.PackFormat.INTERLEAVED)
                        av.append(lo)
                        av.append(hi)
                    for d in range(4):
                        acc = av[0] * bg[0][d]
                        for c8 in range(1, 8):
                            acc = acc + av[c8] * bg[c8][d]
                        plsc.store_scatter(o_buf[b], [ovec + (a * 4 + d)], acc)
            return carry

        lax.fori_loop(0, _NG, group_body, 0)

    def _flush(k, b):
        off = (base + k * _CHUNK) * _OUT
        pltpu.make_async_copy(o_buf[b], out_hbm.at[pl.ds(off, _CHUNK * _OUT)],
                              sem_o[b]).start()

    def _drain(b):
        pltpu.make_async_copy(o_buf[b], out_hbm.at[pl.ds(0, _CHUNK * _OUT)],
                              sem_o[b]).wait()

    def body(i, carry):
        k0 = i * 2
        for bb in range(2):
            k = k0 + bb
            @pl.when(k0 >= 2)
            def _():
                _drain(bb)

            _compute(k, bb)
            _flush(k, bb)
        return carry

    lax.fori_loop(0, _NCHUNK // 2, body, 0)
    _drain(0)
    _drain(1)


def kernel(x, U0, U1, U2, U3):
    t1, t2 = _prep(U0, U1, U2, U3)
    t1p = lax.bitcast_convert_type(
        t1.astype(jnp.bfloat16).reshape(64000, 2), jnp.int32)
    out = _sc_lookup(x.reshape(_N), t1p, t2.reshape(32000))
    return out.reshape(_B, _F, _OUT)


# speedup vs baseline: 10.6329x; 1.2712x over previous
import functools

import jax
import jax.numpy as jnp
from jax import lax
from jax.experimental import pallas as pl
from jax.experimental.pallas import tpu as pltpu
from jax.experimental.pallas import tpu_sc as plsc

_B, _F = 16384, 26
_OUT = 64
_N = _B * _F
_NW = 32
_PER_W = _N // _NW
_CHUNK = 128
_NCHUNK = _PER_W // _CHUNK
_G = 16
_NG = _CHUNK // _G


def _prep_body(u0, u1, u2, u3, t1, t2):
    t1[...] = jnp.dot(u0[...], u1[...], preferred_element_type=jnp.float32)
    t2[...] = jnp.dot(u2[...], u3[...], preferred_element_type=jnp.float32)


def _prep(U0, U1, U2, U3):
    return pl.pallas_call(
        _prep_body,
        out_shape=(
            jax.ShapeDtypeStruct((160, 800), jnp.float32),
            jax.ShapeDtypeStruct((640, 50), jnp.float32),
        ),
    )(U0.reshape(160, 8), U1.reshape(8, 800),
      U2.reshape(640, 8), U3.reshape(8, 50))


_sc_mesh = plsc.VectorSubcoreMesh(core_axis_name="c", subcore_axis_name="s")


@functools.partial(
    pl.kernel,
    out_type=jax.ShapeDtypeStruct((_N * _OUT,), jnp.float32),
    mesh=_sc_mesh,
    scratch_types=[
        pltpu.VMEM((68000,), jnp.int32),
        pltpu.VMEM((32320,), jnp.float32),
        pltpu.VMEM((_CHUNK,), jnp.int32),
        pltpu.VMEM((_CHUNK,), jnp.int32),
        pltpu.VMEM((_CHUNK * 65,), jnp.float32),
        pltpu.VMEM((_CHUNK * _OUT,), jnp.float32),
        pltpu.VMEM((_CHUNK * _OUT,), jnp.float32),
        pltpu.SemaphoreType.DMA,
        pltpu.SemaphoreType.DMA,
        pltpu.SemaphoreType.DMA,
        pltpu.SemaphoreType.DMA,
    ],
    compiler_params=pltpu.CompilerParams(
        needs_layout_passes=False, disable_bounds_checks=True),
)
def _sc_lookup(x_hbm, t1p_hbm, t2_hbm, out_hbm,
               t1p, t2, x_b0, x_b1, stg, o_b0, o_b1,
               sem_x0, sem_x1, sem_o0, sem_o1):
    x_buf = (x_b0, x_b1)
    sem_x = (sem_x0, sem_x1)
    o_buf = (o_b0, o_b1)
    sem_o = (sem_o0, sem_o1)
    wid = lax.axis_index("s") * 2 + lax.axis_index("c")
    base = wid * _PER_W
    iota = lax.iota(jnp.int32, _G)

    pltpu.sync_copy(t1p_hbm, t1p)
    pltpu.sync_copy(t2_hbm, t2)

    def _fetch_x(k, nb):
        pltpu.make_async_copy(x_hbm.at[pl.ds(base + k * _CHUNK, _CHUNK)],
                              x_buf[nb], sem_x[nb]).start()

    def _wait_x(b):
        pltpu.make_async_copy(x_hbm.at[pl.ds(0, _CHUNK)], x_buf[b],
                              sem_x[b]).wait()

    def _fdiv(v, recip, mul):
        q = ((v.astype(jnp.float32) + 0.5) * recip).astype(jnp.int32)
        return q, v - q * mul

    def _compute(b):
        def group_body(g, carry):
            xv = x_buf[b][pl.ds(g * _G, _G)]
            ovec = (iota + g * _G) * 65
            rv, cv = _fdiv(xv, 0.001, 1000)
            n1, n2 = _fdiv(rv, 0.04, 25)
            n3, n4 = _fdiv(cv, 0.04, 25)
            base_a = n1 * 1700 + n2 * 17
            base_b = n3 * 101 + n4 * 2
            bg = [[plsc.load_gather(
                       t2, [base_b + (c8 * 4040 + m3 * 50 + m4)])
                   for m3 in range(2) for m4 in range(2)]
                  for c8 in range(8)]
            for m1 in range(4):
                for m2 in range(4):
                    a = m1 * 4 + m2
                    av = []
                    for cp in range(4):
                        packed = plsc.load_gather(
                            t1p, [base_a + (m1 * 425 + m2 * 4 + cp)])
                        lo, hi = plsc.unpack(
                            plsc.bitcast(packed, jnp.bfloat16),
                            format=plsc.PackFormat.INTERLEAVED)
                        av.append(lo)
                        av.append(hi)
                    for d in range(4):
                        acc = av[0] * bg[0][d]
                        for c8 in range(1, 8):
                            acc = acc + av[c8] * bg[c8][d]
                        plsc.store_scatter(stg, [ovec + (a * 4 + d)], acc)
            return carry

        lax.fori_loop(0, _NG, group_body, 0)

    def _repack(b):
        def rp_body(t, carry):
            for j in range(4):
                o_buf[b][pl.ds(t * _OUT + j * _G, _G)] = (
                    stg[pl.ds(t * 65 + j * _G, _G)])
            return carry

        lax.fori_loop(0, _CHUNK, rp_body, 0)

    def _flush(k, b):
        off = (base + k * _CHUNK) * _OUT
        pltpu.make_async_copy(o_buf[b], out_hbm.at[pl.ds(off, _CHUNK * _OUT)],
                              sem_o[b]).start()

    def _drain(b):
        pltpu.make_async_copy(o_buf[b], out_hbm.at[pl.ds(0, _CHUNK * _OUT)],
                              sem_o[b]).wait()

    _fetch_x(0, 0)

    def body(i, carry):
        k0 = i * 2
        for bb in range(2):
            k = k0 + bb
            nb = (bb + 1) % 2
            if bb == 0:
                _fetch_x(k + 1, nb)
            else:
                @pl.when(k + 1 < _NCHUNK)
                def _():
                    _fetch_x(k + 1, nb)

            _wait_x(bb)
            _compute(bb)

            @pl.when(k0 >= 2)
            def _():
                _drain(bb)

            _repack(bb)
            _flush(k, bb)
        return carry

    lax.fori_loop(0, _NCHUNK // 2, body, 0)
    _drain(0)
    _drain(1)


def kernel(x, U0, U1, U2, U3):
    t1, t2 = _prep(U0, U1, U2, U3)
    t1p = lax.bitcast_convert_type(
        t1.astype(jnp.bfloat16).reshape(64000, 2), jnp.int32)
    t1p = jnp.pad(t1p.reshape(4000, 16), ((0, 0), (0, 1))).reshape(68000)
    t2p = jnp.pad(t2.reshape(320, 100), ((0, 0), (0, 1))).reshape(32320)
    out = _sc_lookup(x.reshape(_N), t1p, t2p)
    return out.reshape(_B, _F, _OUT)
